# baseline (speedup 1.0000x reference)
import functools

import jax
import jax.numpy as jnp
from jax import lax
from jax.experimental import pallas as pl
from jax.experimental.pallas import tpu as pltpu
from jax.experimental.pallas import tpu_sc as plsc

_B, _N, _D = 16, 576, 64
_K = 1024
_BETA = 0.25
_M = _B * _N
_R = 1024
_GRID = _M // _R
_RB = 128
_CB = 128

_NUM_CORES = 2
_NUM_SUBCORES = 16
_NW = _NUM_CORES * _NUM_SUBCORES
_RPW = _M // _NW


def _tc_body(z_ref, w_ref, idx_ref, loss_ref):
    i = pl.program_id(0)
    z = z_ref[...]
    w = w_ref[...]
    zsq = jnp.sum(z * z, axis=1)[None, :]
    esq = jnp.sum(w * w, axis=1, keepdims=True)
    dot2t = lax.dot_general(w + w, z, (((1,), (1,)), ((), ())))

    rowio = lax.broadcasted_iota(jnp.int32, (_CB, 1), 0).astype(jnp.float32)
    step_loss = jnp.zeros((1, 1), jnp.float32)
    for cb in range(_R // _CB):
        c0 = cb * _CB
        zs_b = zsq[:, c0:c0 + _CB]
        acc_v = None
        acc_i = None
        for s in range(_K // _CB):
            s0 = s * _CB
            d_b = (zs_b + esq[s0:s0 + _CB]) - dot2t[s0:s0 + _CB, c0:c0 + _CB]
            si = rowio + float(s0)
            if s == 0:
                acc_v = d_b
                acc_i = jnp.broadcast_to(si, (_CB, _CB))
            else:
                lt = d_b < acc_v
                acc_v = jnp.minimum(acc_v, d_b)
                acc_i = jnp.where(lt, si, acc_i)
        mv = jnp.min(acc_v, axis=0, keepdims=True)
        mi = jnp.min(jnp.where(acc_v == mv, acc_i, float(4 * _K)), axis=0)
        idx_ref[c0:c0 + _CB] = mi.astype(jnp.int32)
        step_loss = step_loss + jnp.sum(mv).reshape(1, 1)

    @pl.when(i == 0)
    def _init():
        loss_ref[...] = jnp.zeros((1, 1), jnp.float32)

    loss_ref[...] += step_loss

    @pl.when(i == _GRID - 1)
    def _finalize():
        s = loss_ref[...] * (1.0 / float(_M * _D))
        loss_ref[...] = s + _BETA * s


def _tc_argmin(zf, w):
    return pl.pallas_call(
        _tc_body,
        grid=(_GRID,),
        in_specs=[
            pl.BlockSpec((_R, _D), lambda i: (i, 0)),
            pl.BlockSpec((_K, _D), lambda i: (0, 0)),
        ],
        out_specs=[
            pl.BlockSpec((_R,), lambda i: (i,)),
            pl.BlockSpec((1, 1), lambda i: (0, 0)),
        ],
        out_shape=[
            jax.ShapeDtypeStruct((_M,), jnp.int32),
            jax.ShapeDtypeStruct((1, 1), jnp.float32),
        ],
    )(zf, w)


def _sc_gather_body(table_hbm, idx_hbm, out_hbm, idx_v, rows_v, sem):
    wid = lax.axis_index("s") * _NUM_CORES + lax.axis_index("c")
    base = wid * _RPW
    pltpu.sync_copy(idx_hbm.at[pl.ds(base, _RPW)], idx_v)
    pltpu.async_copy(table_hbm.at[idx_v], rows_v, sem).wait()
    pltpu.sync_copy(rows_v, out_hbm.at[pl.ds(base, _RPW)])


@functools.cache
def _sc_gather():
    return pl.kernel(
        _sc_gather_body,
        out_type=jax.ShapeDtypeStruct((_M, _D), jnp.float32),
        mesh=plsc.VectorSubcoreMesh(
            core_axis_name="c", subcore_axis_name="s",
            num_cores=_NUM_CORES, num_subcores=_NUM_SUBCORES),
        scratch_types=[
            pltpu.VMEM((_RPW,), jnp.int32),
            pltpu.VMEM((_RPW, _D), jnp.float32),
            pltpu.SemaphoreType.DMA,
        ],
        compiler_params=pltpu.CompilerParams(use_tc_tiling_on_sc=False),
    )


def kernel(z, W):
    zf = z.reshape(_M, _D)
    idx, loss = _tc_argmin(zf, W)
    q = _sc_gather()(W, idx)
    return (q.reshape(_B, _N, _D), loss[0, 0], idx.reshape(_B, _N))


# speedup vs baseline: 27.3627x; 27.3627x over previous
import functools

import jax
import jax.numpy as jnp
from jax import lax
from jax.experimental import pallas as pl
from jax.experimental.pallas import tpu as pltpu
from jax.experimental.pallas import tpu_sc as plsc

_B, _N, _D = 16, 576, 64
_K = 1024
_BETA = 0.25
_M = _B * _N
_R = 1024
_GRID = _M // _R
_RB = 128
_CB = 128

_NUM_CORES = 2
_NUM_SUBCORES = 16
_NW = _NUM_CORES * _NUM_SUBCORES
_RPW = _M // _NW


def _tc_body(z_ref, w_ref, idx_ref, loss_ref):
    i = pl.program_id(0)
    z = z_ref[...]
    w = w_ref[...]
    z_sq = jnp.sum(z * z, axis=1, keepdims=True)
    e_sq = jnp.sum(w * w, axis=1)[None, :]
    dot2 = lax.dot_general(z, w + w, (((1,), (1,)), ((), ())))

    lane_io = lax.broadcasted_iota(jnp.int32, (1, _CB), 1).astype(jnp.float32)
    step_loss = jnp.zeros((1, 1), jnp.float32)
    for rb in range(_R // _RB):
        r0 = rb * _RB
        zs = z_sq[r0:r0 + _RB]
        lane_min = None
        lane_c = None
        for c in range(_K // _CB):
            c0 = c * _CB
            d_c = (zs + e_sq[:, c0:c0 + _CB]) - dot2[r0:r0 + _RB, c0:c0 + _CB]
            if c == 0:
                lane_min = d_c
                lane_c = jnp.zeros((_RB, _CB), jnp.float32)
            else:
                lt = d_c < lane_min
                lane_min = jnp.minimum(lane_min, d_c)
                lane_c = jnp.where(lt, float(c), lane_c)
        m = jnp.min(lane_min, axis=1, keepdims=True)
        packed = jnp.where(lane_min == m, lane_c * float(_CB) + lane_io,
                           float(4 * _K))
        idxf = jnp.min(packed, axis=1)
        idx_ref[r0:r0 + _RB] = idxf.astype(jnp.int32)
        step_loss = step_loss + jnp.sum(m).reshape(1, 1)

    @pl.when(i == 0)
    def _init():
        loss_ref[...] = jnp.zeros((1, 1), jnp.float32)

    loss_ref[...] += step_loss

    @pl.when(i == _GRID - 1)
    def _finalize():
        s = loss_ref[...] * (1.0 / float(_M * _D))
        loss_ref[...] = s + _BETA * s


def _tc_argmin(zf, w):
    return pl.pallas_call(
        _tc_body,
        grid=(_GRID,),
        in_specs=[
            pl.BlockSpec((_R, _D), lambda i: (i, 0)),
            pl.BlockSpec((_K, _D), lambda i: (0, 0)),
        ],
        out_specs=[
            pl.BlockSpec((_R,), lambda i: (i,)),
            pl.BlockSpec((1, 1), lambda i: (0, 0)),
        ],
        out_shape=[
            jax.ShapeDtypeStruct((_M,), jnp.int32),
            jax.ShapeDtypeStruct((1, 1), jnp.float32),
        ],
    )(zf, w)


def _sc_gather_body(table_hbm, idx_hbm, out_hbm, idx_v, rows_v, sem):
    wid = lax.axis_index("s") * _NUM_CORES + lax.axis_index("c")
    base = wid * _RPW
    pltpu.sync_copy(idx_hbm.at[pl.ds(base, _RPW)], idx_v)
    pltpu.async_copy(table_hbm.at[idx_v], rows_v, sem).wait()
    pltpu.sync_copy(rows_v, out_hbm.at[pl.ds(base, _RPW)])


@functools.cache
def _sc_gather():
    return pl.kernel(
        _sc_gather_body,
        out_type=jax.ShapeDtypeStruct((_M, _D), jnp.float32),
        mesh=plsc.VectorSubcoreMesh(
            core_axis_name="c", subcore_axis_name="s",
            num_cores=_NUM_CORES, num_subcores=_NUM_SUBCORES),
        scratch_types=[
            pltpu.VMEM((_RPW,), jnp.int32),
            pltpu.VMEM((_RPW, _D), jnp.float32),
            pltpu.SemaphoreType.DMA,
        ],
        compiler_params=pltpu.CompilerParams(use_tc_tiling_on_sc=False),
    )


def kernel(z, W):
    zf = z.reshape(_M, _D)
    idx, loss = _tc_argmin(zf, W)
    q = _sc_gather()(W, idx)
    return (q.reshape(_B, _N, _D), loss[0, 0], idx.reshape(_B, _N))


# speedup vs baseline: 29.1637x; 1.0658x over previous
import functools

import jax
import jax.numpy as jnp
from jax import lax
from jax.experimental import pallas as pl
from jax.experimental.pallas import tpu as pltpu
from jax.experimental.pallas import tpu_sc as plsc

_B, _N, _D = 16, 576, 64
_K = 1024
_BETA = 0.25
_M = _B * _N
_R = 3072
_GRID = _M // _R
_RB = 128
_CB = 128

_NUM_CORES = 2
_NUM_SUBCORES = 16
_NW = _NUM_CORES * _NUM_SUBCORES
_RPW = _M // _NW


def _tc_body(z_ref, w_ref, idx_ref, loss_ref):
    i = pl.program_id(0)
    z = z_ref[...]
    w = w_ref[...]
    z_sq = jnp.sum(z * z, axis=1, keepdims=True)
    e_sq = jnp.sum(w * w, axis=1)[None, :]
    dot2 = lax.dot_general(z, w + w, (((1,), (1,)), ((), ())))

    lane_io = lax.broadcasted_iota(jnp.int32, (1, _CB), 1).astype(jnp.float32)
    step_loss = jnp.zeros((1, 1), jnp.float32)
    for rb in range(_R // _RB):
        r0 = rb * _RB
        zs = z_sq[r0:r0 + _RB]
        lane_min = None
        lane_c = None
        for c in range(_K // _CB):
            c0 = c * _CB
            d_c = (zs + e_sq[:, c0:c0 + _CB]) - dot2[r0:r0 + _RB, c0:c0 + _CB]
            if c == 0:
                lane_min = d_c
                lane_c = jnp.zeros((_RB, _CB), jnp.float32)
            else:
                lt = d_c < lane_min
                lane_min = jnp.minimum(lane_min, d_c)
                lane_c = jnp.where(lt, float(c), lane_c)
        m = jnp.min(lane_min, axis=1, keepdims=True)
        packed = jnp.where(lane_min == m, lane_c * float(_CB) + lane_io,
                           float(4 * _K))
        idxf = jnp.min(packed, axis=1)
        idx_ref[r0:r0 + _RB] = idxf.astype(jnp.int32)
        step_loss = step_loss + jnp.sum(m).reshape(1, 1)

    @pl.when(i == 0)
    def _init():
        loss_ref[...] = jnp.zeros((1, 1), jnp.float32)

    loss_ref[...] += step_loss

    @pl.when(i == _GRID - 1)
    def _finalize():
        s = loss_ref[...] * (1.0 / float(_M * _D))
        loss_ref[...] = s + _BETA * s


def _tc_argmin(zf, w):
    return pl.pallas_call(
        _tc_body,
        grid=(_GRID,),
        in_specs=[
            pl.BlockSpec((_R, _D), lambda i: (i, 0)),
            pl.BlockSpec((_K, _D), lambda i: (0, 0)),
        ],
        out_specs=[
            pl.BlockSpec((_R,), lambda i: (i,)),
            pl.BlockSpec((1, 1), lambda i: (0, 0)),
        ],
        out_shape=[
            jax.ShapeDtypeStruct((_M,), jnp.int32),
            jax.ShapeDtypeStruct((1, 1), jnp.float32),
        ],
    )(zf, w)


def _sc_gather_body(table_hbm, idx_hbm, out_hbm, idx_v, rows_v, sem):
    wid = lax.axis_index("s") * _NUM_CORES + lax.axis_index("c")
    base = wid * _RPW
    pltpu.sync_copy(idx_hbm.at[pl.ds(base, _RPW)], idx_v)
    pltpu.async_copy(table_hbm.at[idx_v], rows_v, sem).wait()
    pltpu.sync_copy(rows_v, out_hbm.at[pl.ds(base, _RPW)])


@functools.cache
def _sc_gather():
    return pl.kernel(
        _sc_gather_body,
        out_type=jax.ShapeDtypeStruct((_M, _D), jnp.float32),
        mesh=plsc.VectorSubcoreMesh(
            core_axis_name="c", subcore_axis_name="s",
            num_cores=_NUM_CORES, num_subcores=_NUM_SUBCORES),
        scratch_types=[
            pltpu.VMEM((_RPW,), jnp.int32),
            pltpu.VMEM((_RPW, _D), jnp.float32),
            pltpu.SemaphoreType.DMA,
        ],
        compiler_params=pltpu.CompilerParams(use_tc_tiling_on_sc=False),
    )


def kernel(z, W):
    zf = z.reshape(_M, _D)
    idx, loss = _tc_argmin(zf, W)
    q = _sc_gather()(W, idx)
    return (q.reshape(_B, _N, _D), loss[0, 0], idx.reshape(_B, _N))


# speedup vs baseline: 29.5371x; 1.0128x over previous
import functools

import jax
import jax.numpy as jnp
from jax import lax
from jax.experimental import pallas as pl
from jax.experimental.pallas import tpu as pltpu
from jax.experimental.pallas import tpu_sc as plsc

_B, _N, _D = 16, 576, 64
_K = 1024
_BETA = 0.25
_M = _B * _N
_R = 9216
_GRID = _M // _R
_RB = 128
_CB = 128

_NUM_CORES = 2
_NUM_SUBCORES = 16
_NW = _NUM_CORES * _NUM_SUBCORES
_RPW = _M // _NW


def _tc_body(z_ref, w_ref, idx_ref, loss_ref):
    i = pl.program_id(0)
    z = z_ref[...]
    w = w_ref[...]
    z_sq = jnp.sum(z * z, axis=1, keepdims=True)
    e_sq = jnp.sum(w * w, axis=1)[None, :]
    dot2 = lax.dot_general(z, w + w, (((1,), (1,)), ((), ())))

    lane_io = lax.broadcasted_iota(jnp.int32, (1, _CB), 1).astype(jnp.float32)
    step_loss = jnp.zeros((1, 1), jnp.float32)
    for rb in range(_R // _RB):
        r0 = rb * _RB
        zs = z_sq[r0:r0 + _RB]
        lane_min = None
        lane_c = None
        for c in range(_K // _CB):
            c0 = c * _CB
            d_c = (zs + e_sq[:, c0:c0 + _CB]) - dot2[r0:r0 + _RB, c0:c0 + _CB]
            if c == 0:
                lane_min = d_c
                lane_c = jnp.zeros((_RB, _CB), jnp.float32)
            else:
                lt = d_c < lane_min
                lane_min = jnp.minimum(lane_min, d_c)
                lane_c = jnp.where(lt, float(c), lane_c)
        m = jnp.min(lane_min, axis=1, keepdims=True)
        packed = jnp.where(lane_min == m, lane_c * float(_CB) + lane_io,
                           float(4 * _K))
        idxf = jnp.min(packed, axis=1)
        idx_ref[r0:r0 + _RB] = idxf.astype(jnp.int32)
        step_loss = step_loss + jnp.sum(m).reshape(1, 1)

    @pl.when(i == 0)
    def _init():
        loss_ref[...] = jnp.zeros((1, 1), jnp.float32)

    loss_ref[...] += step_loss

    @pl.when(i == _GRID - 1)
    def _finalize():
        s = loss_ref[...] * (1.0 / float(_M * _D))
        loss_ref[...] = s + _BETA * s


def _tc_argmin(zf, w):
    return pl.pallas_call(
        _tc_body,
        grid=(_GRID,),
        in_specs=[
            pl.BlockSpec((_R, _D), lambda i: (i, 0)),
            pl.BlockSpec((_K, _D), lambda i: (0, 0)),
        ],
        out_specs=[
            pl.BlockSpec((_R,), lambda i: (i,)),
            pl.BlockSpec((1, 1), lambda i: (0, 0)),
        ],
        out_shape=[
            jax.ShapeDtypeStruct((_M,), jnp.int32),
            jax.ShapeDtypeStruct((1, 1), jnp.float32),
        ],
    )(zf, w)


def _sc_gather_body(table_hbm, idx_hbm, out_hbm, idx_v, rows_v, sem):
    wid = lax.axis_index("s") * _NUM_CORES + lax.axis_index("c")
    base = wid * _RPW
    pltpu.sync_copy(idx_hbm.at[pl.ds(base, _RPW)], idx_v)
    pltpu.async_copy(table_hbm.at[idx_v], rows_v, sem).wait()
    pltpu.sync_copy(rows_v, out_hbm.at[pl.ds(base, _RPW)])


@functools.cache
def _sc_gather():
    return pl.kernel(
        _sc_gather_body,
        out_type=jax.ShapeDtypeStruct((_M, _D), jnp.float32),
        mesh=plsc.VectorSubcoreMesh(
            core_axis_name="c", subcore_axis_name="s",
            num_cores=_NUM_CORES, num_subcores=_NUM_SUBCORES),
        scratch_types=[
            pltpu.VMEM((_RPW,), jnp.int32),
            pltpu.VMEM((_RPW, _D), jnp.float32),
            pltpu.SemaphoreType.DMA,
        ],
        compiler_params=pltpu.CompilerParams(use_tc_tiling_on_sc=False),
    )


def kernel(z, W):
    zf = z.reshape(_M, _D)
    idx, loss = _tc_argmin(zf, W)
    q = _sc_gather()(W, idx)
    return (q.reshape(_B, _N, _D), loss[0, 0], idx.reshape(_B, _N))
